# Initial kernel scaffold; baseline (speedup 1.0000x reference)
import jax
import jax.numpy as jnp
from jax.experimental import pallas as pl


def kernel(edge_index, h, x, params):
    raise NotImplementedError("write your pallas kernel here")


# speedup vs baseline: 1.0029x; 1.0029x over previous
import functools

import jax
import jax.numpy as jnp
from jax.experimental import pallas as pl
from jax.experimental.pallas import tpu as pltpu

N = 10000
E = 320000
D = 128
BE = 2560


def _edge_body(hi, hj, ci, cj, wcat, wsq, bcat, w2m, b2m, aw, ab, w2x, b2x,
               mm_o, ox_o, ss_o):
    d16 = ci[...] - cj[...]
    sq = jnp.sum(d16 * d16, axis=1, keepdims=True)
    pre = (
        jnp.dot(hi[...], wcat[:D], preferred_element_type=jnp.float32)
        + jnp.dot(hj[...], wcat[D:], preferred_element_type=jnp.float32)
        + sq * wsq[...] + bcat[...]
    )
    s = pre * jax.nn.sigmoid(pre)
    sm = s[:, :D]
    sx = s[:, D:]
    m2 = jnp.dot(sm, w2m[...], preferred_element_type=jnp.float32) + b2m[...]
    att = jax.nn.sigmoid(jnp.sum(m2 * aw[...], axis=1, keepdims=True) + ab[...])
    mm_o[...] = m2 * att
    mx = jnp.sum(sx * w2x[...], axis=1, keepdims=True) + b2x[...]
    ox_o[...] = d16 * mx

    @pl.when(pl.program_id(0) == 0)
    def _():
        ss_o[0, 0] = 0.0

    ss_o[0, 0] += jnp.sum(sq)


@functools.partial(jax.jit, static_argnames=())
def _edge_stage(hi, hj, ci, cj, wcat, wsq, bcat, w2m, b2m, aw, ab, w2x, b2x):
    ne = hi.shape[0]
    grid = (ne // BE,)
    row = lambda shape: pl.BlockSpec(shape, lambda e: (0, 0))
    return pl.pallas_call(
        _edge_body,
        grid=grid,
        in_specs=[
            pl.BlockSpec((BE, D), lambda e: (e, 0)),
            pl.BlockSpec((BE, D), lambda e: (e, 0)),
            pl.BlockSpec((BE, 16), lambda e: (e, 0)),
            pl.BlockSpec((BE, 16), lambda e: (e, 0)),
            row((2 * D, 2 * D)), row((1, 2 * D)), row((1, 2 * D)),
            row((D, D)), row((1, D)), row((1, D)), row((1, 1)),
            row((1, D)), row((1, 1)),
        ],
        out_specs=[
            pl.BlockSpec((BE, D), lambda e: (e, 0)),
            pl.BlockSpec((BE, 16), lambda e: (e, 0)),
            pl.BlockSpec(memory_space=pltpu.SMEM),
        ],
        out_shape=[
            jax.ShapeDtypeStruct((ne, D), jnp.float32),
            jax.ShapeDtypeStruct((ne, 16), jnp.float32),
            jax.ShapeDtypeStruct((1, 1), jnp.float32),
        ],
    )(hi, hj, ci, cj, wcat, wsq, bcat, w2m, b2m, aw, ab, w2x, b2x)


def _node_body(h, smsg, c16, agg3, ss, w1h, b1h, w2h, b2h, h_o, c_o):
    t = (
        jnp.dot(h[...], w1h[:D], preferred_element_type=jnp.float32)
        + jnp.dot(smsg[...], w1h[D:], preferred_element_type=jnp.float32)
        + b1h[...]
    )
    t = t * jax.nn.sigmoid(t)
    h_o[...] = jnp.dot(t, w2h[...], preferred_element_type=jnp.float32) + b2h[...]
    scale = 1.0 / (jnp.sqrt(ss[0, 0]) + 1.0)
    c_o[...] = c16[...] + agg3[...] * scale


def _node_stage(h, smsg, c16, agg3, ss, w1h, b1h, w2h, b2h):
    v = pl.BlockSpec(memory_space=pltpu.VMEM)
    return pl.pallas_call(
        _node_body,
        in_specs=[v, v, v, v, pl.BlockSpec(memory_space=pltpu.SMEM),
                  v, v, v, v],
        out_shape=[
            jax.ShapeDtypeStruct((N, D), jnp.float32),
            jax.ShapeDtypeStruct((N, 16), jnp.float32),
        ],
    )(h, smsg, c16, agg3, ss, w1h, b1h, w2h, b2h)


def kernel(edge_index, h, x, params):
    i = edge_index[0]
    j = edge_index[1]
    c16 = jnp.pad(x, ((0, 0), (0, 13)))
    for p in params:
        w1m, b1m = p['m'][0]
        w2m, b2m = p['m'][1]
        w1x, b1x = p['x'][0]
        w2x, b2x = p['x'][1]
        aw, ab = p['att']
        w1h, b1h = p['h'][0]
        w2h, b2h = p['h'][1]
        wcat = jnp.concatenate([w1m[:2 * D], w1x[:2 * D]], axis=1)
        wsq = jnp.concatenate([w1m[2 * D], w1x[2 * D]], axis=0)[None, :]
        bcat = jnp.concatenate([b1m, b1x], axis=0)[None, :]

        hi = h[i]
        hj = h[j]
        ci = c16[i]
        cj = c16[j]
        mm, ox, ss = _edge_stage(
            hi, hj, ci, cj, wcat, wsq, bcat, w2m, b2m[None, :],
            aw[:, 0][None, :], ab[None, :], w2x[:, 0][None, :], b2x[None, :])
        smsg = jax.ops.segment_sum(mm, i, num_segments=N)
        agg3 = jax.ops.segment_sum(ox, i, num_segments=N)
        h, c16 = _node_stage(h, smsg, c16, agg3, ss,
                             w1h, b1h[None, :], w2h, b2h[None, :])
    return (h, c16[:, :3])


# speedup vs baseline: 4.2261x; 4.2140x over previous
import functools

import jax
import jax.numpy as jnp
from jax import lax
from jax.experimental import pallas as pl
from jax.experimental.pallas import tpu as pltpu
from jax.experimental.pallas import tpu_sc as plsc

N = 10000
E = 320000
D = 128
BE = 2560
GW = 128

_SC_MESH = plsc.VectorSubcoreMesh(core_axis_name="c", subcore_axis_name="s")


def _sc_gather(h, c16, i2d, j2d):
    @functools.partial(
        pl.kernel,
        out_type=[
            jax.ShapeDtypeStruct((E, D), jnp.float32),
            jax.ShapeDtypeStruct((E, D), jnp.float32),
            jax.ShapeDtypeStruct((E, 16), jnp.float32),
            jax.ShapeDtypeStruct((E, 16), jnp.float32),
        ],
        mesh=_SC_MESH,
        compiler_params=pltpu.CompilerParams(use_tc_tiling_on_sc=False),
    )
    def k(h_hbm, c_hbm, i_hbm, j_hbm, hi_hbm, hj_hbm, ci_hbm, cj_hbm):
        def body(i_v, j_v, hi_o, hj_o, ci_o, cj_o):
            pltpu.sync_copy(h_hbm.at[i_v.at[0]], hi_o)
            pltpu.sync_copy(h_hbm.at[j_v.at[0]], hj_o)
            pltpu.sync_copy(c_hbm.at[i_v.at[0]], ci_o)
            pltpu.sync_copy(c_hbm.at[j_v.at[0]], cj_o)

        pltpu.emit_pipeline(
            body,
            grid=(E // GW,),
            in_specs=[
                pl.BlockSpec((1, GW), lambda g: (0, g)),
                pl.BlockSpec((1, GW), lambda g: (0, g)),
            ],
            out_specs=[
                pl.BlockSpec((GW, D), lambda g: (g, 0)),
                pl.BlockSpec((GW, D), lambda g: (g, 0)),
                pl.BlockSpec((GW, 16), lambda g: (g, 0)),
                pl.BlockSpec((GW, 16), lambda g: (g, 0)),
            ],
            core_axis_name=("c", "s"),
            dimension_semantics=(pltpu.PARALLEL,),
        )(i_hbm, j_hbm, hi_hbm, hj_hbm, ci_hbm, cj_hbm)

    return k(h, c16, i2d, j2d)


SB = 80


def _sc_scatter(i2d, mm, ox, z128, z16):
    @functools.partial(
        pl.kernel,
        out_type=[
            jax.ShapeDtypeStruct((2, N, D), jnp.float32),
            jax.ShapeDtypeStruct((2, N, 16), jnp.float32),
        ],
        mesh=_SC_MESH,
        scratch_types=[
            pltpu.VMEM_SHARED((N, D), jnp.float32),
            pltpu.VMEM_SHARED((N, 16), jnp.float32),
        ],
        compiler_params=pltpu.CompilerParams(use_tc_tiling_on_sc=False),
    )
    def k(i_hbm, mm_hbm, ox_hbm, z128_hbm, z16_hbm, om_hbm, o3_hbm, accm, acc3):
        c = lax.axis_index("c")
        s = lax.axis_index("s")
        rows = N // 16
        r0 = s * rows
        pltpu.sync_copy(z128_hbm.at[pl.ds(r0, rows)], accm.at[pl.ds(r0, rows)])
        pltpu.sync_copy(z16_hbm.at[pl.ds(r0, rows)], acc3.at[pl.ds(r0, rows)])
        plsc.subcore_barrier()

        def body(i_v, mm_v, ox_v):
            pltpu.sync_copy(mm_v, accm.at[i_v.at[0]], add=True)
            pltpu.sync_copy(ox_v, acc3.at[i_v.at[0]], add=True)

        pltpu.emit_pipeline(
            body,
            grid=(E // SB,),
            in_specs=[
                pl.BlockSpec((1, SB), lambda g: (0, g)),
                pl.BlockSpec((SB, D), lambda g: (g, 0)),
                pl.BlockSpec((SB, 16), lambda g: (g, 0)),
            ],
            out_specs=[],
            core_axis_name=("c", "s"),
            dimension_semantics=(pltpu.PARALLEL,),
        )(i_hbm, mm_hbm, ox_hbm)

        plsc.subcore_barrier()
        pltpu.sync_copy(accm.at[pl.ds(r0, rows)], om_hbm.at[c, pl.ds(r0, rows)])
        pltpu.sync_copy(acc3.at[pl.ds(r0, rows)], o3_hbm.at[c, pl.ds(r0, rows)])

    return k(i2d, mm, ox, z128, z16)


def _edge_body(hi, hj, ci, cj, wcat, wsq, bcat, w2m, b2m, aw, ab, w2x, b2x,
               mm_o, ox_o, ss_o):
    d16 = ci[...] - cj[...]
    sq = jnp.sum(d16 * d16, axis=1, keepdims=True)
    pre = (
        jnp.dot(hi[...], wcat[:D], preferred_element_type=jnp.float32)
        + jnp.dot(hj[...], wcat[D:], preferred_element_type=jnp.float32)
        + sq * wsq[...] + bcat[...]
    )
    s = pre * jax.nn.sigmoid(pre)
    sm = s[:, :D]
    sx = s[:, D:]
    m2 = jnp.dot(sm, w2m[...], preferred_element_type=jnp.float32) + b2m[...]
    att = jax.nn.sigmoid(jnp.sum(m2 * aw[...], axis=1, keepdims=True) + ab[...])
    mm_o[...] = m2 * att
    mx = jnp.sum(sx * w2x[...], axis=1, keepdims=True) + b2x[...]
    ox_o[...] = d16 * mx

    @pl.when(pl.program_id(0) == 0)
    def _():
        ss_o[0, 0] = 0.0

    ss_o[0, 0] += jnp.sum(sq)


@functools.partial(jax.jit, static_argnames=())
def _edge_stage(hi, hj, ci, cj, wcat, wsq, bcat, w2m, b2m, aw, ab, w2x, b2x):
    ne = hi.shape[0]
    grid = (ne // BE,)
    row = lambda shape: pl.BlockSpec(shape, lambda e: (0, 0))
    return pl.pallas_call(
        _edge_body,
        grid=grid,
        in_specs=[
            pl.BlockSpec((BE, D), lambda e: (e, 0)),
            pl.BlockSpec((BE, D), lambda e: (e, 0)),
            pl.BlockSpec((BE, 16), lambda e: (e, 0)),
            pl.BlockSpec((BE, 16), lambda e: (e, 0)),
            row((2 * D, 2 * D)), row((1, 2 * D)), row((1, 2 * D)),
            row((D, D)), row((1, D)), row((1, D)), row((1, 1)),
            row((1, D)), row((1, 1)),
        ],
        out_specs=[
            pl.BlockSpec((BE, D), lambda e: (e, 0)),
            pl.BlockSpec((BE, 16), lambda e: (e, 0)),
            pl.BlockSpec(memory_space=pltpu.SMEM),
        ],
        out_shape=[
            jax.ShapeDtypeStruct((ne, D), jnp.float32),
            jax.ShapeDtypeStruct((ne, 16), jnp.float32),
            jax.ShapeDtypeStruct((1, 1), jnp.float32),
        ],
    )(hi, hj, ci, cj, wcat, wsq, bcat, w2m, b2m, aw, ab, w2x, b2x)


def _node_body(h, smsg, c16, agg3, ss, w1h, b1h, w2h, b2h, h_o, c_o):
    sm = smsg[0] + smsg[1]
    t = (
        jnp.dot(h[...], w1h[:D], preferred_element_type=jnp.float32)
        + jnp.dot(sm, w1h[D:], preferred_element_type=jnp.float32)
        + b1h[...]
    )
    t = t * jax.nn.sigmoid(t)
    h_o[...] = jnp.dot(t, w2h[...], preferred_element_type=jnp.float32) + b2h[...]
    scale = 1.0 / (jnp.sqrt(ss[0, 0]) + 1.0)
    c_o[...] = c16[...] + (agg3[0] + agg3[1]) * scale


def _node_stage(h, smsg, c16, agg3, ss, w1h, b1h, w2h, b2h):
    v = pl.BlockSpec(memory_space=pltpu.VMEM)
    return pl.pallas_call(
        _node_body,
        in_specs=[v, v, v, v, pl.BlockSpec(memory_space=pltpu.SMEM),
                  v, v, v, v],
        out_shape=[
            jax.ShapeDtypeStruct((N, D), jnp.float32),
            jax.ShapeDtypeStruct((N, 16), jnp.float32),
        ],
    )(h, smsg, c16, agg3, ss, w1h, b1h, w2h, b2h)


def kernel(edge_index, h, x, params):
    i = edge_index[0]
    j = edge_index[1]
    i2d = i.reshape(1, E)
    j2d = j.reshape(1, E)
    c16 = jnp.pad(x, ((0, 0), (0, 13)))
    for p in params:
        w1m, b1m = p['m'][0]
        w2m, b2m = p['m'][1]
        w1x, b1x = p['x'][0]
        w2x, b2x = p['x'][1]
        aw, ab = p['att']
        w1h, b1h = p['h'][0]
        w2h, b2h = p['h'][1]
        wcat = jnp.concatenate([w1m[:2 * D], w1x[:2 * D]], axis=1)
        wsq = jnp.concatenate([w1m[2 * D], w1x[2 * D]], axis=0)[None, :]
        bcat = jnp.concatenate([b1m, b1x], axis=0)[None, :]

        hi, hj, ci, cj = _sc_gather(h, c16, i2d, j2d)
        mm, ox, ss = _edge_stage(
            hi, hj, ci, cj, wcat, wsq, bcat, w2m, b2m[None, :],
            aw[:, 0][None, :], ab[None, :], w2x[:, 0][None, :], b2x[None, :])
        z128 = jnp.zeros((N, D), jnp.float32)
        z16 = jnp.zeros((N, 16), jnp.float32)
        smsg, agg3 = _sc_scatter(i2d, mm, ox, z128, z16)
        h, c16 = _node_stage(h, smsg, c16, agg3, ss,
                             w1h, b1h[None, :], w2h, b2h[None, :])
    return (h, c16[:, :3])


# speedup vs baseline: 6.4124x; 1.5173x over previous
import functools

import jax
import jax.numpy as jnp
from jax import lax
from jax.experimental import pallas as pl
from jax.experimental.pallas import tpu as pltpu
from jax.experimental.pallas import tpu_sc as plsc

N = 10000
E = 320000
D = 128
BE = 3200
GW = 128
K = 4

_SC_MESH = plsc.VectorSubcoreMesh(core_axis_name="c", subcore_axis_name="s")


def _sc_gather(tab, i2d, j2d):
    ec = i2d.shape[1]

    @functools.partial(
        pl.kernel,
        out_type=[
            jax.ShapeDtypeStruct((ec, D), jnp.float32),
            jax.ShapeDtypeStruct((ec, D), jnp.float32),
        ],
        mesh=_SC_MESH,
        compiler_params=pltpu.CompilerParams(use_tc_tiling_on_sc=False),
    )
    def k(t_hbm, i_hbm, j_hbm, ti_hbm, tj_hbm):
        def body(i_v, j_v, ti_o, tj_o):
            pltpu.sync_copy(t_hbm.at[i_v.at[0]], ti_o)
            pltpu.sync_copy(t_hbm.at[j_v.at[0]], tj_o)

        pltpu.emit_pipeline(
            body,
            grid=(ec // GW,),
            in_specs=[
                pl.BlockSpec((1, GW), lambda g: (0, g)),
                pl.BlockSpec((1, GW), lambda g: (0, g)),
            ],
            out_specs=[
                pl.BlockSpec((GW, D), lambda g: (g, 0)),
                pl.BlockSpec((GW, D), lambda g: (g, 0)),
            ],
            core_axis_name=("c", "s"),
            dimension_semantics=(pltpu.PARALLEL,),
        )(i_hbm, j_hbm, ti_hbm, tj_hbm)

    return k(tab, i2d, j2d)


def _unpack_bf16(t64):
    w = lax.bitcast_convert_type(t64, jnp.uint32)
    lo = lax.bitcast_convert_type((w & 0xFFFF).astype(jnp.uint16), jnp.bfloat16)
    hi = lax.bitcast_convert_type((w >> 16).astype(jnp.uint16), jnp.bfloat16)
    return jnp.concatenate([lo, hi], axis=1)


def _pack_bf16(h):
    a = lax.bitcast_convert_type(h[:, :64].astype(jnp.bfloat16), jnp.uint16)
    b = lax.bitcast_convert_type(h[:, 64:].astype(jnp.bfloat16), jnp.uint16)
    w = a.astype(jnp.uint32) | (b.astype(jnp.uint32) << 16)
    return lax.bitcast_convert_type(w, jnp.float32)


SB = 80


def _sc_scatter(i2d, mm, ox, z128, z16):
    ec = i2d.shape[1]

    @functools.partial(
        pl.kernel,
        out_type=[
            jax.ShapeDtypeStruct((2, N, D), jnp.float32),
            jax.ShapeDtypeStruct((2, N, 16), jnp.float32),
        ],
        mesh=_SC_MESH,
        scratch_types=[
            pltpu.VMEM_SHARED((N, D), jnp.float32),
            pltpu.VMEM_SHARED((N, 16), jnp.float32),
        ],
        compiler_params=pltpu.CompilerParams(use_tc_tiling_on_sc=False),
    )
    def k(i_hbm, mm_hbm, ox_hbm, z128_hbm, z16_hbm, om_hbm, o3_hbm, accm, acc3):
        c = lax.axis_index("c")
        s = lax.axis_index("s")
        rows = N // 16
        r0 = s * rows
        pltpu.sync_copy(z128_hbm.at[pl.ds(r0, rows)], accm.at[pl.ds(r0, rows)])
        pltpu.sync_copy(z16_hbm.at[pl.ds(r0, rows)], acc3.at[pl.ds(r0, rows)])
        plsc.subcore_barrier()

        def body(i_v, mm_v, ox_v):
            pltpu.sync_copy(mm_v, accm.at[i_v.at[0]], add=True)
            pltpu.sync_copy(ox_v, acc3.at[i_v.at[0]], add=True)

        pltpu.emit_pipeline(
            body,
            grid=(ec // SB,),
            in_specs=[
                pl.BlockSpec((1, SB), lambda g: (0, g)),
                pl.BlockSpec((SB, D), lambda g: (g, 0)),
                pl.BlockSpec((SB, 16), lambda g: (g, 0)),
            ],
            out_specs=[],
            core_axis_name=("c", "s"),
            dimension_semantics=(pltpu.PARALLEL,),
        )(i_hbm, mm_hbm, ox_hbm)

        plsc.subcore_barrier()
        pltpu.sync_copy(accm.at[pl.ds(r0, rows)], om_hbm.at[c, pl.ds(r0, rows)])
        pltpu.sync_copy(acc3.at[pl.ds(r0, rows)], o3_hbm.at[c, pl.ds(r0, rows)])

    return k(i2d, mm, ox, z128, z16)


def _edge_body(ti, tj, wcat, wsq, bcat, w2m, b2m, aw, ab, w2x, b2x,
               mm_o, ox_o, ss_o):
    hi = _unpack_bf16(ti[:, :64])
    hj = _unpack_bf16(tj[:, :64])
    ci = ti[:, 64:80]
    cj = tj[:, 64:80]
    d16 = ci - cj
    sq = jnp.sum(d16 * d16, axis=1, keepdims=True)
    pre = (
        jnp.dot(hi, wcat[:D], preferred_element_type=jnp.float32)
        + jnp.dot(hj, wcat[D:], preferred_element_type=jnp.float32)
        + sq * wsq[...] + bcat[...]
    )
    s = pre * jax.nn.sigmoid(pre)
    sm = s[:, :D]
    sx = s[:, D:]
    m2 = jnp.dot(sm, w2m[...], preferred_element_type=jnp.float32) + b2m[...]
    att = jax.nn.sigmoid(jnp.sum(m2 * aw[...], axis=1, keepdims=True) + ab[...])
    mm_o[...] = m2 * att
    mx = jnp.sum(sx * w2x[...], axis=1, keepdims=True) + b2x[...]
    ox_o[...] = d16 * mx

    @pl.when(pl.program_id(0) == 0)
    def _():
        ss_o[0, 0] = 0.0

    ss_o[0, 0] += jnp.sum(sq)


@functools.partial(jax.jit, static_argnames=())
def _edge_stage(ti, tj, wcat, wsq, bcat, w2m, b2m, aw, ab, w2x, b2x):
    ne = ti.shape[0]
    grid = (ne // BE,)
    row = lambda shape: pl.BlockSpec(shape, lambda e: (0, 0))
    return pl.pallas_call(
        _edge_body,
        grid=grid,
        in_specs=[
            pl.BlockSpec((BE, D), lambda e: (e, 0)),
            pl.BlockSpec((BE, D), lambda e: (e, 0)),
            row((2 * D, 2 * D)), row((1, 2 * D)), row((1, 2 * D)),
            row((D, D)), row((1, D)), row((1, D)), row((1, 1)),
            row((1, D)), row((1, 1)),
        ],
        out_specs=[
            pl.BlockSpec((BE, D), lambda e: (e, 0)),
            pl.BlockSpec((BE, 16), lambda e: (e, 0)),
            pl.BlockSpec(memory_space=pltpu.SMEM),
        ],
        out_shape=[
            jax.ShapeDtypeStruct((ne, D), jnp.float32),
            jax.ShapeDtypeStruct((ne, 16), jnp.float32),
            jax.ShapeDtypeStruct((1, 1), jnp.float32),
        ],
    )(ti, tj, wcat, wsq, bcat, w2m, b2m, aw, ab, w2x, b2x)


BN = 2000


def _node_body(nk, args):
    h, c16 = args[0], args[1]
    smsgs = args[2:2 + nk]
    agg3s = args[2 + nk:2 + 2 * nk]
    sss = args[2 + 2 * nk:2 + 3 * nk]
    w1h, b1h, w2h, b2h = args[2 + 3 * nk:2 + 3 * nk + 4]
    h_o, c_o, t_o = args[-3], args[-2], args[-1]
    sm = smsgs[0][0] + smsgs[0][1]
    for r in smsgs[1:]:
        sm = sm + r[0] + r[1]
    a3 = agg3s[0][0] + agg3s[0][1]
    for r in agg3s[1:]:
        a3 = a3 + r[0] + r[1]
    t = (
        jnp.dot(h[...], w1h[:D], preferred_element_type=jnp.float32)
        + jnp.dot(sm, w1h[D:], preferred_element_type=jnp.float32)
        + b1h[...]
    )
    t = t * jax.nn.sigmoid(t)
    h2 = jnp.dot(t, w2h[...], preferred_element_type=jnp.float32) + b2h[...]
    h_o[...] = h2
    ss = sss[0][0, 0]
    for r in sss[1:]:
        ss = ss + r[0, 0]
    scale = 1.0 / (jnp.sqrt(ss) + 1.0)
    c2 = c16[...] + a3 * scale
    c_o[...] = c2
    t_o[...] = jnp.concatenate(
        [_pack_bf16(h2), c2, jnp.zeros((h2.shape[0], 48), jnp.float32)], axis=1)


def _node_stage(h, c16, smsgs, agg3s, sss, w1h, b1h, w2h, b2h):
    nk = len(smsgs)
    nblk = lambda w: pl.BlockSpec((BN, w), lambda n: (n, 0))
    pblk = lambda w: pl.BlockSpec((2, BN, w), lambda n: (0, n, 0))
    smem = pl.BlockSpec(memory_space=pltpu.SMEM)
    row = lambda shape: pl.BlockSpec(shape, lambda n: (0, 0))
    body = lambda *args: _node_body(nk, args)
    return pl.pallas_call(
        body,
        grid=(N // BN,),
        in_specs=([nblk(D), nblk(16)] + [pblk(D)] * nk + [pblk(16)] * nk
                  + [smem] * nk
                  + [row((2 * D, D)), row((1, D)), row((D, D)), row((1, D))]),
        out_specs=[nblk(D), nblk(16), nblk(D)],
        out_shape=[
            jax.ShapeDtypeStruct((N, D), jnp.float32),
            jax.ShapeDtypeStruct((N, 16), jnp.float32),
            jax.ShapeDtypeStruct((N, D), jnp.float32),
        ],
    )(h, c16, *smsgs, *agg3s, *sss, w1h, b1h, w2h, b2h)


def kernel(edge_index, h, x, params):
    i = edge_index[0]
    j = edge_index[1]
    i2d = i.reshape(1, E)
    j2d = j.reshape(1, E)
    c16 = jnp.pad(x, ((0, 0), (0, 13)))
    tab = jnp.concatenate(
        [_pack_bf16(h), c16, jnp.zeros((N, 48), jnp.float32)], axis=1)
    for p in params:
        w1m, b1m = p['m'][0]
        w2m, b2m = p['m'][1]
        w1x, b1x = p['x'][0]
        w2x, b2x = p['x'][1]
        aw, ab = p['att']
        w1h, b1h = p['h'][0]
        w2h, b2h = p['h'][1]
        wcat = jnp.concatenate([w1m[:2 * D], w1x[:2 * D]], axis=1).astype(jnp.bfloat16)
        wsq = jnp.concatenate([w1m[2 * D], w1x[2 * D]], axis=0)[None, :]
        bcat = jnp.concatenate([b1m, b1x], axis=0)[None, :]

        z128 = jnp.zeros((N, D), jnp.float32)
        z16 = jnp.zeros((N, 16), jnp.float32)
        ec = E // K
        smsgs, agg3s, sss = [], [], []
        for kk in range(K):
            ik = lax.slice(i2d, (0, kk * ec), (1, (kk + 1) * ec))
            jk = lax.slice(j2d, (0, kk * ec), (1, (kk + 1) * ec))
            ti, tj = _sc_gather(tab, ik, jk)
            mm, ox, ss = _edge_stage(
                ti, tj, wcat, wsq, bcat, w2m, b2m[None, :],
                aw[:, 0][None, :], ab[None, :], w2x[:, 0][None, :],
                b2x[None, :])
            smsg, agg3 = _sc_scatter(ik, mm, ox, z128, z16)
            smsgs.append(smsg)
            agg3s.append(agg3)
            sss.append(ss)
        h, c16, tab = _node_stage(h, c16, smsgs, agg3s, sss,
                                  w1h, b1h[None, :], w2h, b2h[None, :])
    return (h, c16[:, :3])


# speedup vs baseline: 6.4575x; 1.0070x over previous
import functools

import jax
import jax.numpy as jnp
from jax import lax
from jax.experimental import pallas as pl
from jax.experimental.pallas import tpu as pltpu
from jax.experimental.pallas import tpu_sc as plsc

N = 10000
E = 320000
D = 128
BE = 3200

GW = 128
K = 5

_SC_MESH = plsc.VectorSubcoreMesh(core_axis_name="c", subcore_axis_name="s")


def _sc_gather(tab, i2d, j2d):
    ec = i2d.shape[1]

    @functools.partial(
        pl.kernel,
        out_type=[
            jax.ShapeDtypeStruct((ec, D), jnp.float32),
            jax.ShapeDtypeStruct((ec, D), jnp.float32),
        ],
        mesh=_SC_MESH,
        compiler_params=pltpu.CompilerParams(use_tc_tiling_on_sc=False),
    )
    def k(t_hbm, i_hbm, j_hbm, ti_hbm, tj_hbm):
        def body(i_v, j_v, ti_o, tj_o):
            pltpu.sync_copy(t_hbm.at[i_v.at[0]], ti_o)
            pltpu.sync_copy(t_hbm.at[j_v.at[0]], tj_o)

        pltpu.emit_pipeline(
            body,
            grid=(ec // GW,),
            in_specs=[
                pl.BlockSpec((1, GW), lambda g: (0, g)),
                pl.BlockSpec((1, GW), lambda g: (0, g)),
            ],
            out_specs=[
                pl.BlockSpec((GW, D), lambda g: (g, 0)),
                pl.BlockSpec((GW, D), lambda g: (g, 0)),
            ],
            core_axis_name=("c", "s"),
            dimension_semantics=(pltpu.PARALLEL,),
        )(i_hbm, j_hbm, ti_hbm, tj_hbm)

    return k(tab, i2d, j2d)


def _unpack_bf16(t64):
    w = lax.bitcast_convert_type(t64, jnp.uint32)
    lo = lax.bitcast_convert_type((w & 0xFFFF).astype(jnp.uint16), jnp.bfloat16)
    hi = lax.bitcast_convert_type((w >> 16).astype(jnp.uint16), jnp.bfloat16)
    return jnp.concatenate([lo, hi], axis=1)


def _pack_bf16(h):
    a = lax.bitcast_convert_type(h[:, :64].astype(jnp.bfloat16), jnp.uint16)
    b = lax.bitcast_convert_type(h[:, 64:].astype(jnp.bfloat16), jnp.uint16)
    w = a.astype(jnp.uint32) | (b.astype(jnp.uint32) << 16)
    return lax.bitcast_convert_type(w, jnp.float32)


SB = 80


def _sc_scatter(i2d, mm, ox, z128, z16):
    ec = i2d.shape[1]

    @functools.partial(
        pl.kernel,
        out_type=[
            jax.ShapeDtypeStruct((2, N, D), jnp.float32),
            jax.ShapeDtypeStruct((2, N, 16), jnp.float32),
        ],
        mesh=_SC_MESH,
        scratch_types=[
            pltpu.VMEM_SHARED((N, D), jnp.float32),
            pltpu.VMEM_SHARED((N, 16), jnp.float32),
        ],
        compiler_params=pltpu.CompilerParams(use_tc_tiling_on_sc=False),
    )
    def k(i_hbm, mm_hbm, ox_hbm, z128_hbm, z16_hbm, om_hbm, o3_hbm, accm, acc3):
        c = lax.axis_index("c")
        s = lax.axis_index("s")
        rows = N // 16
        r0 = s * rows
        pltpu.sync_copy(z128_hbm.at[pl.ds(r0, rows)], accm.at[pl.ds(r0, rows)])
        pltpu.sync_copy(z16_hbm.at[pl.ds(r0, rows)], acc3.at[pl.ds(r0, rows)])
        plsc.subcore_barrier()

        def body(i_v, mm_v, ox_v):
            pltpu.sync_copy(mm_v, accm.at[i_v.at[0]], add=True)
            pltpu.sync_copy(ox_v, acc3.at[i_v.at[0]], add=True)

        pltpu.emit_pipeline(
            body,
            grid=(ec // SB,),
            in_specs=[
                pl.BlockSpec((1, SB), lambda g: (0, g)),
                pl.BlockSpec((SB, D), lambda g: (g, 0)),
                pl.BlockSpec((SB, 16), lambda g: (g, 0)),
            ],
            out_specs=[],
            core_axis_name=("c", "s"),
            dimension_semantics=(pltpu.PARALLEL,),
        )(i_hbm, mm_hbm, ox_hbm)

        plsc.subcore_barrier()
        pltpu.sync_copy(accm.at[pl.ds(r0, rows)], om_hbm.at[c, pl.ds(r0, rows)])
        pltpu.sync_copy(acc3.at[pl.ds(r0, rows)], o3_hbm.at[c, pl.ds(r0, rows)])

    return k(i2d, mm, ox, z128, z16)


def _edge_body(ti, tj, wcat, wsq, bcat, w2m, b2m, aw, ab, w2x, b2x,
               mm_o, ox_o, ss_o):
    hi = _unpack_bf16(ti[:, :64])
    hj = _unpack_bf16(tj[:, :64])
    ci = ti[:, 64:80]
    cj = tj[:, 64:80]
    d16 = ci - cj
    sq = jnp.sum(d16 * d16, axis=1, keepdims=True)
    pre = (
        jnp.dot(hi, wcat[:D], preferred_element_type=jnp.float32)
        + jnp.dot(hj, wcat[D:], preferred_element_type=jnp.float32)
        + sq * wsq[...] + bcat[...]
    )
    s = pre * jax.nn.sigmoid(pre)
    sm = s[:, :D]
    sx = s[:, D:]
    m2 = jnp.dot(sm, w2m[...], preferred_element_type=jnp.float32) + b2m[...]
    att = jax.nn.sigmoid(jnp.sum(m2 * aw[...], axis=1, keepdims=True) + ab[...])
    mm_o[...] = m2 * att
    mx = jnp.sum(sx * w2x[...], axis=1, keepdims=True) + b2x[...]
    ox_o[...] = d16 * mx

    @pl.when(pl.program_id(0) == 0)
    def _():
        ss_o[0, 0] = 0.0

    ss_o[0, 0] += jnp.sum(sq)


@functools.partial(jax.jit, static_argnames=())
def _edge_stage(ti, tj, wcat, wsq, bcat, w2m, b2m, aw, ab, w2x, b2x):
    ne = ti.shape[0]
    grid = (ne // BE,)
    row = lambda shape: pl.BlockSpec(shape, lambda e: (0, 0))
    return pl.pallas_call(
        _edge_body,
        grid=grid,
        in_specs=[
            pl.BlockSpec((BE, D), lambda e: (e, 0)),
            pl.BlockSpec((BE, D), lambda e: (e, 0)),
            row((2 * D, 2 * D)), row((1, 2 * D)), row((1, 2 * D)),
            row((D, D)), row((1, D)), row((1, D)), row((1, 1)),
            row((1, D)), row((1, 1)),
        ],
        out_specs=[
            pl.BlockSpec((BE, D), lambda e: (e, 0)),
            pl.BlockSpec((BE, 16), lambda e: (e, 0)),
            pl.BlockSpec(memory_space=pltpu.SMEM),
        ],
        out_shape=[
            jax.ShapeDtypeStruct((ne, D), jnp.float32),
            jax.ShapeDtypeStruct((ne, 16), jnp.float32),
            jax.ShapeDtypeStruct((1, 1), jnp.float32),
        ],
    )(ti, tj, wcat, wsq, bcat, w2m, b2m, aw, ab, w2x, b2x)


BN = 2000


def _node_body(nk, args):
    h, c16 = args[0], args[1]
    smsgs = args[2:2 + nk]
    agg3s = args[2 + nk:2 + 2 * nk]
    sss = args[2 + 2 * nk:2 + 3 * nk]
    w1h, b1h, w2h, b2h = args[2 + 3 * nk:2 + 3 * nk + 4]
    h_o, c_o, t_o = args[-3], args[-2], args[-1]
    sm = smsgs[0][0] + smsgs[0][1]
    for r in smsgs[1:]:
        sm = sm + r[0] + r[1]
    a3 = agg3s[0][0] + agg3s[0][1]
    for r in agg3s[1:]:
        a3 = a3 + r[0] + r[1]
    t = (
        jnp.dot(h[...], w1h[:D], preferred_element_type=jnp.float32)
        + jnp.dot(sm, w1h[D:], preferred_element_type=jnp.float32)
        + b1h[...]
    )
    t = t * jax.nn.sigmoid(t)
    h2 = jnp.dot(t, w2h[...], preferred_element_type=jnp.float32) + b2h[...]
    h_o[...] = h2
    ss = sss[0][0, 0]
    for r in sss[1:]:
        ss = ss + r[0, 0]
    scale = 1.0 / (jnp.sqrt(ss) + 1.0)
    c2 = c16[...] + a3 * scale
    c_o[...] = c2
    t_o[...] = jnp.concatenate(
        [_pack_bf16(h2), c2, jnp.zeros((h2.shape[0], 48), jnp.float32)], axis=1)


def _node_stage(h, c16, smsgs, agg3s, sss, w1h, b1h, w2h, b2h):
    nk = len(smsgs)
    nblk = lambda w: pl.BlockSpec((BN, w), lambda n: (n, 0))
    pblk = lambda w: pl.BlockSpec((2, BN, w), lambda n: (0, n, 0))
    smem = pl.BlockSpec(memory_space=pltpu.SMEM)
    row = lambda shape: pl.BlockSpec(shape, lambda n: (0, 0))
    body = lambda *args: _node_body(nk, args)
    return pl.pallas_call(
        body,
        grid=(N // BN,),
        in_specs=([nblk(D), nblk(16)] + [pblk(D)] * nk + [pblk(16)] * nk
                  + [smem] * nk
                  + [row((2 * D, D)), row((1, D)), row((D, D)), row((1, D))]),
        out_specs=[nblk(D), nblk(16), nblk(D)],
        out_shape=[
            jax.ShapeDtypeStruct((N, D), jnp.float32),
            jax.ShapeDtypeStruct((N, 16), jnp.float32),
            jax.ShapeDtypeStruct((N, D), jnp.float32),
        ],
    )(h, c16, *smsgs, *agg3s, *sss, w1h, b1h, w2h, b2h)


def kernel(edge_index, h, x, params):
    i = edge_index[0]
    j = edge_index[1]
    i2d = i.reshape(1, E)
    j2d = j.reshape(1, E)
    c16 = jnp.pad(x, ((0, 0), (0, 13)))
    tab = jnp.concatenate(
        [_pack_bf16(h), c16, jnp.zeros((N, 48), jnp.float32)], axis=1)
    for p in params:
        w1m, b1m = p['m'][0]
        w2m, b2m = p['m'][1]
        w1x, b1x = p['x'][0]
        w2x, b2x = p['x'][1]
        aw, ab = p['att']
        w1h, b1h = p['h'][0]
        w2h, b2h = p['h'][1]
        wcat = jnp.concatenate([w1m[:2 * D], w1x[:2 * D]], axis=1).astype(jnp.bfloat16)
        wsq = jnp.concatenate([w1m[2 * D], w1x[2 * D]], axis=0)[None, :]
        bcat = jnp.concatenate([b1m, b1x], axis=0)[None, :]

        z128 = jnp.zeros((N, D), jnp.float32)
        z16 = jnp.zeros((N, 16), jnp.float32)
        ec = E // K
        smsgs, agg3s, sss = [], [], []
        for kk in range(K):
            ik = lax.slice(i2d, (0, kk * ec), (1, (kk + 1) * ec))
            jk = lax.slice(j2d, (0, kk * ec), (1, (kk + 1) * ec))
            ti, tj = _sc_gather(tab, ik, jk)
            mm, ox, ss = _edge_stage(
                ti, tj, wcat, wsq, bcat, w2m, b2m[None, :],
                aw[:, 0][None, :], ab[None, :], w2x[:, 0][None, :],
                b2x[None, :])
            smsg, agg3 = _sc_scatter(ik, mm, ox, z128, z16)
            smsgs.append(smsg)
            agg3s.append(agg3)
            sss.append(ss)
        h, c16, tab = _node_stage(h, c16, smsgs, agg3s, sss,
                                  w1h, b1h[None, :], w2h, b2h[None, :])
    return (h, c16[:, :3])
